# Initial kernel scaffold; baseline (speedup 1.0000x reference)
import jax
import jax.numpy as jnp
from jax.experimental import pallas as pl


def kernel(x, table):
    raise NotImplementedError("write your pallas kernel here")


# speedup vs baseline: 3.3417x; 3.3417x over previous
import jax
import jax.numpy as jnp
from jax.experimental import pallas as pl


_BLOCK_ROWS = 512


def _copy_block(table_ref, out_ref):
    out_ref[...] = table_ref[...]


def kernel(x, table):
    seq_len = x.shape[-1]
    hidden = table.shape[1]
    num_blocks = seq_len // _BLOCK_ROWS
    return pl.pallas_call(
        _copy_block,
        grid=(num_blocks,),
        in_specs=[pl.BlockSpec((_BLOCK_ROWS, hidden), lambda i: (i, 0))],
        out_specs=pl.BlockSpec((_BLOCK_ROWS, hidden), lambda i: (i, 0)),
        out_shape=jax.ShapeDtypeStruct((seq_len, hidden), table.dtype),
    )(table)
